# baseline (speedup 1.0000x reference)
import functools

import jax
import jax.numpy as jnp
from jax import lax
from jax.experimental import pallas as pl
from jax.experimental.pallas import tpu as pltpu
from jax.experimental.pallas import tpu_sc as plsc

VOCAB = 1000000
EMBED_DIM = 64
BATCH = 16384
HIST_LEN = 50

NUM_CORES = 2
NUM_SUBCORES = 16
NUM_WORKERS = NUM_CORES * NUM_SUBCORES

B_FLAT = BATCH * HIST_LEN
B_PER_W = B_FLAT // NUM_WORKERS
PAD_DIM = 128
CHUNK = 256
NCHUNKS = B_PER_W // CHUNK
NPAIRS = NCHUNKS // 2

assert B_PER_W * NUM_WORKERS == B_FLAT
assert NPAIRS * 2 * CHUNK == B_PER_W

_MESH = plsc.VectorSubcoreMesh(
    core_axis_name="c",
    subcore_axis_name="s",
    num_cores=NUM_CORES,
    num_subcores=NUM_SUBCORES,
)


@functools.partial(
    pl.kernel,
    out_type=jax.ShapeDtypeStruct((B_FLAT, EMBED_DIM), jnp.float32),
    mesh=_MESH,
    scratch_types=[
        pltpu.VMEM((CHUNK,), jnp.int32),
        pltpu.VMEM((CHUNK,), jnp.int32),
        pltpu.VMEM((CHUNK, PAD_DIM), jnp.float32),
        pltpu.VMEM((CHUNK, PAD_DIM), jnp.float32),
        pltpu.SemaphoreType.DMA,
        pltpu.SemaphoreType.DMA,
        pltpu.SemaphoreType.DMA,
        pltpu.SemaphoreType.DMA,
        pltpu.SemaphoreType.DMA,
        pltpu.SemaphoreType.DMA,
    ],
    compiler_params=pltpu.CompilerParams(use_tc_tiling_on_sc=False),
)
def _gather_kernel(table_hbm, idx_hbm, out_hbm,
                   idx0, idx1, rows0, rows1,
                   si0, si1, sg0, sg1, so0, so1):
    wid = lax.axis_index("s") * NUM_CORES + lax.axis_index("c")
    base = wid * B_PER_W

    def pair_body(c, first, last):
        g0 = pltpu.async_copy(table_hbm.at[idx0], rows0, sg0)
        if not first:
            pltpu.make_async_copy(
                rows1.at[:, pl.ds(0, EMBED_DIM)],
                out_hbm.at[pl.ds(base + (c - 1) * CHUNK, CHUNK)], so1
            ).wait()
        g0.wait()
        o0 = pltpu.async_copy(
            rows0.at[:, pl.ds(0, EMBED_DIM)],
            out_hbm.at[pl.ds(base + c * CHUNK, CHUNK)], so0)
        if not last:
            i0 = pltpu.async_copy(
                idx_hbm.at[pl.ds(base + (c + 2) * CHUNK, CHUNK)], idx0, si0)
        g1 = pltpu.async_copy(table_hbm.at[idx1], rows1, sg1)
        g1.wait()
        o1 = pltpu.async_copy(
            rows1.at[:, pl.ds(0, EMBED_DIM)],
            out_hbm.at[pl.ds(base + (c + 1) * CHUNK, CHUNK)], so1)
        if not last:
            i1 = pltpu.async_copy(
                idx_hbm.at[pl.ds(base + (c + 3) * CHUNK, CHUNK)], idx1, si1)
            i0.wait()
            i1.wait()
        o0.wait()
        if last:
            o1.wait()

    pltpu.sync_copy(idx_hbm.at[pl.ds(base, CHUNK)], idx0)
    pltpu.sync_copy(idx_hbm.at[pl.ds(base + CHUNK, CHUNK)], idx1)

    pair_body(0, first=True, last=False)

    @pl.loop(1, NPAIRS - 1)
    def _pair(i):
        pair_body(2 * i, first=False, last=False)

    pair_body(2 * (NPAIRS - 1), first=False, last=True)


def kernel(item_embeddings, batch_data):
    idx = batch_data.reshape(-1).astype(jnp.int32)
    table128 = jnp.pad(item_embeddings, ((0, 0), (0, PAD_DIM - EMBED_DIM)))
    out = _gather_kernel(table128, idx)
    return out.reshape(BATCH, HIST_LEN, EMBED_DIM)


# speedup vs baseline: 1.1219x; 1.1219x over previous
import functools

import jax
import jax.numpy as jnp
from jax import lax
from jax.experimental import pallas as pl
from jax.experimental.pallas import tpu as pltpu
from jax.experimental.pallas import tpu_sc as plsc

VOCAB = 1000000
EMBED_DIM = 64
BATCH = 16384
HIST_LEN = 50

NUM_CORES = 2
NUM_SUBCORES = 16
NUM_WORKERS = NUM_CORES * NUM_SUBCORES

B_FLAT = BATCH * HIST_LEN
B_PER_W = B_FLAT // NUM_WORKERS
CHUNK = 512
NCHUNKS = B_PER_W // CHUNK
NPAIRS = NCHUNKS // 2

assert B_PER_W * NUM_WORKERS == B_FLAT
assert NPAIRS * 2 * CHUNK == B_PER_W

_MESH = plsc.VectorSubcoreMesh(
    core_axis_name="c",
    subcore_axis_name="s",
    num_cores=NUM_CORES,
    num_subcores=NUM_SUBCORES,
)


@functools.partial(
    pl.kernel,
    out_type=jax.ShapeDtypeStruct((B_FLAT, EMBED_DIM), jnp.float32),
    mesh=_MESH,
    scratch_types=[
        pltpu.VMEM((CHUNK,), jnp.int32),
        pltpu.VMEM((CHUNK,), jnp.int32),
        pltpu.VMEM((CHUNK, EMBED_DIM), jnp.float32),
        pltpu.VMEM((CHUNK, EMBED_DIM), jnp.float32),
        pltpu.SemaphoreType.DMA,
        pltpu.SemaphoreType.DMA,
        pltpu.SemaphoreType.DMA,
        pltpu.SemaphoreType.DMA,
        pltpu.SemaphoreType.DMA,
        pltpu.SemaphoreType.DMA,
    ],
    compiler_params=pltpu.CompilerParams(use_tc_tiling_on_sc=False),
)
def _gather_kernel(table_hbm, idx_hbm, out_hbm,
                   idx0, idx1, rows0, rows1,
                   si0, si1, sg0, sg1, so0, so1):
    wid = lax.axis_index("s") * NUM_CORES + lax.axis_index("c")
    base = wid * B_PER_W

    def pair_body(c, first, last):
        g0 = pltpu.async_copy(table_hbm.at[idx0], rows0, sg0)
        if not first:
            pltpu.make_async_copy(
                rows1, out_hbm.at[pl.ds(base + (c - 1) * CHUNK, CHUNK)], so1
            ).wait()
        g0.wait()
        o0 = pltpu.async_copy(
            rows0, out_hbm.at[pl.ds(base + c * CHUNK, CHUNK)], so0)
        if not last:
            i0 = pltpu.async_copy(
                idx_hbm.at[pl.ds(base + (c + 2) * CHUNK, CHUNK)], idx0, si0)
        g1 = pltpu.async_copy(table_hbm.at[idx1], rows1, sg1)
        g1.wait()
        o1 = pltpu.async_copy(
            rows1, out_hbm.at[pl.ds(base + (c + 1) * CHUNK, CHUNK)], so1)
        if not last:
            i1 = pltpu.async_copy(
                idx_hbm.at[pl.ds(base + (c + 3) * CHUNK, CHUNK)], idx1, si1)
            i0.wait()
            i1.wait()
        o0.wait()
        if last:
            o1.wait()

    pltpu.sync_copy(idx_hbm.at[pl.ds(base, CHUNK)], idx0)
    pltpu.sync_copy(idx_hbm.at[pl.ds(base + CHUNK, CHUNK)], idx1)

    pair_body(0, first=True, last=False)

    @pl.loop(1, NPAIRS - 1)
    def _pair(i):
        pair_body(2 * i, first=False, last=False)

    pair_body(2 * (NPAIRS - 1), first=False, last=True)


def kernel(item_embeddings, batch_data):
    idx = batch_data.reshape(-1).astype(jnp.int32)
    out = _gather_kernel(item_embeddings, idx)
    return out.reshape(BATCH, HIST_LEN, EMBED_DIM)


# speedup vs baseline: 1.2834x; 1.1439x over previous
import functools

import jax
import jax.numpy as jnp
from jax import lax
from jax.experimental import pallas as pl
from jax.experimental.pallas import tpu as pltpu
from jax.experimental.pallas import tpu_sc as plsc

VOCAB = 1000000
EMBED_DIM = 64
BATCH = 16384
HIST_LEN = 50

NUM_CORES = 2
NUM_SUBCORES = 16
NUM_WORKERS = NUM_CORES * NUM_SUBCORES

B_FLAT = BATCH * HIST_LEN
B_PER_W = B_FLAT // NUM_WORKERS
CHUNK = 512
NCHUNKS = B_PER_W // CHUNK
NPAIRS = NCHUNKS // 2

assert B_PER_W * NUM_WORKERS == B_FLAT
assert NPAIRS * 2 * CHUNK == B_PER_W

_MESH = plsc.VectorSubcoreMesh(
    core_axis_name="c",
    subcore_axis_name="s",
    num_cores=NUM_CORES,
    num_subcores=NUM_SUBCORES,
)


@functools.partial(
    pl.kernel,
    out_type=jax.ShapeDtypeStruct((B_FLAT, EMBED_DIM), jnp.float32),
    mesh=_MESH,
    scratch_types=[
        pltpu.VMEM((CHUNK,), jnp.int32),
        pltpu.VMEM((CHUNK,), jnp.int32),
        pltpu.VMEM((CHUNK, EMBED_DIM), jnp.float32),
        pltpu.VMEM((CHUNK, EMBED_DIM), jnp.float32),
        pltpu.SemaphoreType.DMA,
        pltpu.SemaphoreType.DMA,
        pltpu.SemaphoreType.DMA,
        pltpu.SemaphoreType.DMA,
        pltpu.SemaphoreType.DMA,
        pltpu.SemaphoreType.DMA,
    ],
    compiler_params=pltpu.CompilerParams(use_tc_tiling_on_sc=False),
)
def _gather_kernel(table_hbm, idx_hbm, out_hbm,
                   idx0, idx1, rows0, rows1,
                   si0, si1, sg0, sg1, so0, so1):
    wid = lax.axis_index("s") * NUM_CORES + lax.axis_index("c")
    base = wid * B_PER_W

    def pair_body(c, first, last):
        g0 = pltpu.async_copy(table_hbm.at[idx0], rows0, sg0)
        if not first:
            pltpu.make_async_copy(
                rows1, out_hbm.at[pl.ds(base + (c - 1) * CHUNK, CHUNK)], so1
            ).wait()
        g0.wait()
        o0 = pltpu.async_copy(
            rows0, out_hbm.at[pl.ds(base + c * CHUNK, CHUNK)], so0)
        if not last:
            i0 = pltpu.async_copy(
                idx_hbm.at[pl.ds(base + (c + 2) * CHUNK, CHUNK)], idx0, si0)
        g1 = pltpu.async_copy(table_hbm.at[idx1], rows1, sg1)
        g1.wait()
        o1 = pltpu.async_copy(
            rows1, out_hbm.at[pl.ds(base + (c + 1) * CHUNK, CHUNK)], so1)
        if not last:
            i1 = pltpu.async_copy(
                idx_hbm.at[pl.ds(base + (c + 3) * CHUNK, CHUNK)], idx1, si1)
            i0.wait()
            i1.wait()
        o0.wait()
        if last:
            o1.wait()

    pltpu.sync_copy(idx_hbm.at[pl.ds(base, CHUNK)], idx0)
    pltpu.sync_copy(idx_hbm.at[pl.ds(base + CHUNK, CHUNK)], idx1)

    pair_body(0, first=True, last=False)

    @pl.loop(1, NPAIRS - 1)
    def _pair(i):
        pair_body(2 * i, first=False, last=False)

    pair_body(2 * (NPAIRS - 1), first=False, last=True)


def kernel(item_embeddings, batch_data):
    idx = batch_data.reshape(-1).astype(jnp.int32)
    out = _gather_kernel(item_embeddings, idx)
    o2 = jax.lax.optimization_barrier(
        out.reshape(BATCH, HIST_LEN * EMBED_DIM))
    return o2.reshape(BATCH, HIST_LEN, EMBED_DIM)
